# Initial kernel scaffold; baseline (speedup 1.0000x reference)
import jax
import jax.numpy as jnp
from jax.experimental import pallas as pl


def kernel(x, expert_weights, w_gate, w_up, w_down, expert_indices, top_k):
    raise NotImplementedError("write your pallas kernel here")


# speedup vs baseline: 1.6725x; 1.6725x over previous
import functools

import jax
import jax.numpy as jnp
from jax import lax
from jax.experimental import pallas as pl
from jax.experimental.pallas import tpu as pltpu
from jax.experimental.pallas import tpu_sc as plsc


def _sc_gather_rows(table, idx):
    info = plsc.get_sparse_core_info()
    nc, ns = info.num_cores, info.num_subcores
    nw = nc * ns
    b, = idx.shape
    d = table.shape[1]
    b_per_w = b // nw
    mesh = plsc.VectorSubcoreMesh(core_axis_name="c", subcore_axis_name="s")

    @functools.partial(
        pl.kernel, mesh=mesh,
        out_type=jax.ShapeDtypeStruct((b, d), table.dtype),
        scratch_types=[
            pltpu.VMEM((b_per_w,), jnp.int32),
            pltpu.VMEM((b_per_w, d), table.dtype),
            pltpu.SemaphoreType.DMA,
        ],
    )
    def k(table_hbm, idx_hbm, out_hbm, idx_v, rows_v, sem):
        wid = lax.axis_index("s") * nc + lax.axis_index("c")
        base = wid * b_per_w
        pltpu.sync_copy(idx_hbm.at[pl.ds(base, b_per_w)], idx_v)
        pltpu.async_copy(table_hbm.at[idx_v], rows_v, sem).wait()
        pltpu.sync_copy(rows_v, out_hbm.at[pl.ds(base, b_per_w)])

    return k(table, idx)


def _sc_combine_rows(rows_sorted, invperm, n_tokens, k_per_token):
    info = plsc.get_sparse_core_info()
    nc, ns = info.num_cores, info.num_subcores
    nw = nc * ns
    d = rows_sorted.shape[1]
    t_per_w = n_tokens // nw
    rows_per_w = t_per_w * k_per_token
    lanes = info.num_lanes
    mesh = plsc.VectorSubcoreMesh(core_axis_name="c", subcore_axis_name="s")

    @functools.partial(
        pl.kernel, mesh=mesh,
        out_type=jax.ShapeDtypeStruct((n_tokens, d), rows_sorted.dtype),
        scratch_types=[
            pltpu.VMEM((rows_per_w,), jnp.int32),
            pltpu.VMEM((rows_per_w, d), rows_sorted.dtype),
            pltpu.VMEM((t_per_w, d), rows_sorted.dtype),
            pltpu.SemaphoreType.DMA,
        ],
    )
    def k(rows_hbm, inv_hbm, out_hbm, idx_v, rows_v, out_v, sem):
        wid = lax.axis_index("s") * nc + lax.axis_index("c")
        pltpu.sync_copy(inv_hbm.at[pl.ds(wid * rows_per_w, rows_per_w)], idx_v)
        pltpu.async_copy(rows_hbm.at[idx_v], rows_v, sem).wait()

        def body(i, carry):
            for c in range(d // lanes):
                sl = pl.ds(c * lanes, lanes)
                acc = rows_v[i * k_per_token, sl]
                for kk in range(1, k_per_token):
                    acc = acc + rows_v[i * k_per_token + kk, sl]
                out_v[i, sl] = acc
            return carry

        lax.fori_loop(0, t_per_w, body, 0)
        pltpu.sync_copy(out_v, out_hbm.at[pl.ds(wid * t_per_w, t_per_w)])

    return k(rows_sorted, invperm)


_BM = 256


def _grouped_mm_kernel(meta_ref, xs_ref, ws_ref, wg_ref, wu_ref, wd_ref,
                       out_ref):
    t = pl.program_id(0)
    lo = meta_ref[2, t]
    hi = meta_ref[3, t]

    @pl.when(meta_ref[4, t] == 1)
    def _init():
        out_ref[...] = jnp.zeros_like(out_ref)

    @pl.when(hi > lo)
    def _compute():
        rows = lax.broadcasted_iota(jnp.int32, (xs_ref.shape[0], 1), 0)
        mask = (rows >= lo) & (rows < hi)
        xb = jnp.where(mask, xs_ref[...], 0.0)
        g = jnp.dot(xb, wg_ref[0], preferred_element_type=jnp.float32)
        u = jnp.dot(xb, wu_ref[0], preferred_element_type=jnp.float32)
        h = g * lax.logistic(g) * u * ws_ref[...]
        out_ref[...] += jnp.dot(h, wd_ref[0], preferred_element_type=jnp.float32)


def _grouped_mm(xs_sorted, ws_sorted, w_gate, w_up, w_down, meta, n_tiles):
    nk, d = xs_sorted.shape
    inter = w_gate.shape[2]
    bm = _BM
    grid_spec = pltpu.PrefetchScalarGridSpec(
        num_scalar_prefetch=1,
        grid=(n_tiles,),
        in_specs=[
            pl.BlockSpec((bm, d), lambda t, m: (m[0, t], 0)),
            pl.BlockSpec((bm, 1), lambda t, m: (m[0, t], 0)),
            pl.BlockSpec((1, d, inter), lambda t, m: (m[1, t], 0, 0)),
            pl.BlockSpec((1, d, inter), lambda t, m: (m[1, t], 0, 0)),
            pl.BlockSpec((1, inter, d), lambda t, m: (m[1, t], 0, 0)),
        ],
        out_specs=pl.BlockSpec((bm, d), lambda t, m: (m[0, t], 0)),
    )
    return pl.pallas_call(
        _grouped_mm_kernel,
        grid_spec=grid_spec,
        out_shape=jax.ShapeDtypeStruct((nk, d), xs_sorted.dtype),
        compiler_params=pltpu.CompilerParams(
            dimension_semantics=("arbitrary",),
        ),
    )(meta, xs_sorted, ws_sorted, w_gate, w_up, w_down)


def _routing_metadata(expert_indices, expert_weights, n_experts, bm, n_tiles):
    n, k = expert_indices.shape
    nk = n * k
    nb = nk // bm
    i32 = jnp.int32
    flat_e = expert_indices.reshape(-1).astype(i32)
    order = jnp.argsort(flat_e, stable=True).astype(i32)
    sorted_e = flat_e[order]
    token_ids = order // k
    invperm = jnp.argsort(order).astype(i32)
    ws_sorted = expert_weights.reshape(-1)[order].reshape(nk, 1)

    counts = jnp.bincount(flat_e, length=n_experts)
    off = jnp.concatenate(
        [jnp.zeros((1,), i32), jnp.cumsum(counts).astype(i32)])
    first_e = sorted_e[::bm]
    last_e = sorted_e[bm - 1::bm]
    tiles_pb = last_e - first_e + 1
    cum = jnp.cumsum(tiles_pb)
    cumx = cum - tiles_pb
    t_ids = jnp.arange(n_tiles, dtype=i32)
    blk = jnp.searchsorted(cum, t_ids, side="right").astype(i32)
    blk_c = jnp.minimum(blk, nb - 1)
    e_t = jnp.clip(first_e[blk_c] + (t_ids - cumx[blk_c]), 0, n_experts - 1)
    valid = t_ids < cum[-1]
    lo = jnp.clip(jnp.maximum(off[e_t], blk_c * bm) - blk_c * bm, 0, bm)
    hi = jnp.clip(jnp.minimum(off[e_t + 1], (blk_c + 1) * bm) - blk_c * bm,
                  0, bm)
    lo = jnp.where(valid, lo, 0)
    hi = jnp.where(valid, hi, 0)
    first = ((t_ids == cumx[blk_c]) & valid).astype(i32)
    meta = jnp.stack([blk_c, e_t, lo, hi, first])
    return token_ids, invperm, ws_sorted, meta


def kernel(x, expert_weights, w_gate, w_up, w_down, expert_indices, top_k):
    n, d = x.shape
    e_num = w_gate.shape[0]
    k = expert_indices.shape[1]
    nk = n * k
    bm = _BM
    n_tiles = nk // bm + e_num - 1

    token_ids, invperm, ws_sorted, meta = _routing_metadata(
        expert_indices, expert_weights, e_num, bm, n_tiles)

    xs_sorted = _sc_gather_rows(x, token_ids)
    down_sorted = _grouped_mm(
        xs_sorted, ws_sorted.astype(x.dtype), w_gate, w_up, w_down, meta,
        n_tiles)
    return _sc_combine_rows(down_sorted, invperm, n, k)
